# baseline (speedup 1.0000x reference)
import functools

import jax
import jax.numpy as jnp
from jax.experimental import pallas as pl
from jax.experimental.pallas import tpu as pltpu

N = 10000
D = 128
M_BLK = 400


def _gin_block(eps_ref, adj_ref, feat_ref, feat_row_ref, out_ref):
    scale = 1.0 + eps_ref[0]
    neigh = jnp.dot(adj_ref[...], feat_ref[...],
                    preferred_element_type=jnp.float32)
    out_ref[...] = scale * feat_row_ref[...] + neigh


@jax.jit
def kernel(adj, feat, eps):
    grid = (N // M_BLK,)
    return pl.pallas_call(
        _gin_block,
        grid=grid,
        in_specs=[
            pl.BlockSpec(memory_space=pltpu.SMEM),
            pl.BlockSpec((M_BLK, N), lambda i: (i, 0)),
            pl.BlockSpec((N, D), lambda i: (0, 0)),
            pl.BlockSpec((M_BLK, D), lambda i: (i, 0)),
        ],
        out_specs=pl.BlockSpec((M_BLK, D), lambda i: (i, 0)),
        out_shape=jax.ShapeDtypeStruct((N, D), jnp.float32),
    )(eps, adj, feat, feat)
